# baseline (speedup 1.0000x reference)
import functools

import jax
import jax.numpy as jnp
from jax import lax
from jax.experimental import pallas as pl
from jax.experimental.pallas import tpu as pltpu
from jax.experimental.pallas import tpu_sc as plsc

B = 16384
D = 32
L = 16
PACK = 128 // D
CH = 2

_MESH = dict(core_axis_name="c", subcore_axis_name="s")


def _user_gather(user_ids, uembT):
    info = plsc.get_sparse_core_info()
    nc, ns = info.num_cores, info.num_subcores
    bpw = B // (nc * ns)

    @functools.partial(
        pl.kernel,
        mesh=plsc.VectorSubcoreMesh(**_MESH),
        compiler_params=pltpu.CompilerParams(
            needs_layout_passes=False, disable_bounds_checks=True),
        out_type=jax.ShapeDtypeStruct((B * D,), jnp.float32),
        scratch_types=[
            pltpu.VMEM((bpw,), jnp.int32),
            *[pltpu.VMEM((D, 128), jnp.float32) for _ in range(L)],
            pltpu.VMEM((bpw * D,), jnp.float32),
            pltpu.SemaphoreType.DMA,
        ],
    )
    def ka(uids_hbm, uembT_hbm, out_hbm, uidx, *rest):
        ubuf = rest[:L]
        urows, usem = rest[L:]
        wid = lax.axis_index("s") * nc + lax.axis_index("c")
        base = wid * bpw

        pltpu.sync_copy(uids_hbm.at[pl.ds(base, bpw)], uidx)
        dlo = lax.iota(jnp.int32, L)
        dhi = dlo + L
        ngrp = bpw // L

        def wait_one(r):
            pltpu.make_async_copy(
                uembT_hbm.at[pl.ds(0, D), pl.ds(0, 128)], ubuf[r], usem).wait()

        def extract(uvec, g, r):
            ul = jnp.full((L,), uvec[r] & 127, jnp.int32)
            i = g * L + r
            urows[pl.ds(i * D, L)] = plsc.load_gather(ubuf[r], [dlo, ul])
            urows[pl.ds(i * D + L, L)] = plsc.load_gather(ubuf[r], [dhi, ul])

        def group(g, _):
            cur = uidx[pl.ds(g * L, L)]
            prv = uidx[pl.ds(jnp.maximum(g - 1, 0) * L, L)]
            for r in range(L):
                @pl.when(g > 0)
                def _():
                    wait_one(r)
                    extract(prv, g - 1, r)

                blk = pl.multiple_of((cur[r] >> 7) * 128, 128)
                pltpu.async_copy(
                    uembT_hbm.at[pl.ds(0, D), pl.ds(blk, 128)], ubuf[r], usem)
            return 0

        lax.fori_loop(0, ngrp, group, 0)
        lastv = uidx[pl.ds((ngrp - 1) * L, L)]
        for r in range(L):
            wait_one(r)
            extract(lastv, ngrp - 1, r)
        pltpu.sync_copy(urows, out_hbm.at[pl.ds(base * D, bpw * D)])

    return ka(user_ids, uembT)


def _dot_bias(user_rows_flat, user_ids, movie_ids, memb, ub1d, mb1d):
    info = plsc.get_sparse_core_info()
    nc, ns = info.num_cores, info.num_subcores
    bpw = B // (nc * ns)

    @functools.partial(
        pl.kernel,
        mesh=plsc.VectorSubcoreMesh(**_MESH),
        compiler_params=pltpu.CompilerParams(needs_layout_passes=False),
        out_type=jax.ShapeDtypeStruct((B,), jnp.float32),
        scratch_types=[
            pltpu.VMEM((bpw,), jnp.int32),
            pltpu.VMEM((bpw,), jnp.int32),
            *[pltpu.VMEM((8, D), jnp.float32) for _ in range(2 * L)],
            pltpu.VMEM((bpw * D,), jnp.float32),
            pltpu.VMEM((bpw,), jnp.float32),
            pltpu.VMEM((bpw,), jnp.float32),
            pltpu.VMEM((bpw,), jnp.float32),
            pltpu.SemaphoreType.DMA,
            pltpu.SemaphoreType.DMA,
        ],
    )
    def kb(urows_hbm, uids_hbm, mids_hbm, memb_hbm, ub_hbm, mb_hbm, out_hbm,
           uidx, midx, *rest):
        mbuf = rest[:2 * L]
        urows, ubias, mbias, outv, sem, msem = rest[2 * L:]
        wid = lax.axis_index("s") * nc + lax.axis_index("c")
        base = wid * bpw

        pltpu.sync_copy(uids_hbm.at[pl.ds(base, bpw)], uidx)
        pltpu.sync_copy(mids_hbm.at[pl.ds(base, bpw)], midx)
        cu = pltpu.async_copy(
            urows_hbm.at[pl.ds(base * D, bpw * D)], urows, sem)
        cb1 = pltpu.async_copy(ub_hbm.at[uidx], ubias, sem)
        cb2 = pltpu.async_copy(mb_hbm.at[midx], mbias, sem)
        lane = lax.iota(jnp.int32, L)
        ngrp = bpw // L
        cu.wait()

        def wait_slab(r):
            pltpu.make_async_copy(
                memb_hbm.at[pl.ds(0, 8), pl.ds(0, D)], mbuf[r], msem).wait()

        def dot_one(mvec, i, rb, rl):
            ms = mvec[rl] & 7
            m0 = mbuf[rb][ms, pl.ds(0, L)]
            m1 = mbuf[rb][ms, pl.ds(L, L)]
            u0 = urows[pl.ds(i * D, L)]
            u1 = urows[pl.ds(i * D + L, L)]
            s = u0 * m0 + u1 * m1
            return jnp.sum(s)

        G2 = 2 * L
        ngrp2 = bpw // G2

        def half_dots(prv_lo, prv_hi, gp, off):
            acc_lo = jnp.zeros((L,), jnp.float32)
            acc_hi = jnp.zeros((L,), jnp.float32)
            for r in range(G2):
                if off:
                    wait_slab(r)
                if r < L:
                    acc_lo = acc_lo + jnp.where(
                        lane == r, dot_one(prv_lo, gp * G2 + r, r, r), 0.0)
                else:
                    acc_hi = acc_hi + jnp.where(
                        lane == (r - L),
                        dot_one(prv_hi, gp * G2 + r, r, r - L), 0.0)
            outv[pl.ds(gp * G2, L)] = acc_lo
            outv[pl.ds(gp * G2 + L, L)] = acc_hi

        def group(g, _):
            gp = jnp.maximum(g - 1, 0)
            prv_lo = midx[pl.ds(gp * G2, L)]
            prv_hi = midx[pl.ds(gp * G2 + L, L)]
            cur_lo = midx[pl.ds(g * G2, L)]
            cur_hi = midx[pl.ds(g * G2 + L, L)]

            @pl.when(g > 0)
            def _():
                half_dots(prv_lo, prv_hi, gp, True)

            for r in range(G2):
                mr = cur_lo[r] if r < L else cur_hi[r - L]
                row8 = pl.multiple_of((mr >> 3) * 8, 8)
                pltpu.async_copy(
                    memb_hbm.at[pl.ds(row8, 8), pl.ds(0, D)], mbuf[r], msem)
            return 0

        lax.fori_loop(0, ngrp2, group, 0)
        last_lo = midx[pl.ds((ngrp2 - 1) * G2, L)]
        last_hi = midx[pl.ds((ngrp2 - 1) * G2 + L, L)]
        half_dots(last_lo, last_hi, ngrp2 - 1, True)

        cb1.wait()
        cb2.wait()

        def badd(j, _):
            sl = pl.ds(j * L, L)
            outv[sl] = outv[sl] + mbias[sl] + ubias[sl]
            return 0

        lax.fori_loop(0, bpw // L, badd, 0, unroll=4)

        pltpu.sync_copy(outv, out_hbm.at[pl.ds(base, bpw)])

    return kb(user_rows_flat, user_ids, movie_ids, memb, ub1d, mb1d)


def kernel(user_ids, movie_ids, user_emb, movie_emb, user_bias, movie_bias):
    n_movies, _ = movie_emb.shape
    uids = user_ids.astype(jnp.int32)
    mids = movie_ids.astype(jnp.int32)
    urows_flat = _user_gather(uids, user_emb.T)
    return _dot_bias(
        urows_flat,
        uids,
        mids,
        movie_emb,
        user_bias.reshape(-1),
        movie_bias.reshape(-1),
    )


# speedup vs baseline: 1.0320x; 1.0320x over previous
import functools

import jax
import jax.numpy as jnp
from jax import lax
from jax.experimental import pallas as pl
from jax.experimental.pallas import tpu as pltpu
from jax.experimental.pallas import tpu_sc as plsc

B = 16384
D = 32
L = 16
PACK = 128 // D
CH = 2

_MESH = dict(core_axis_name="c", subcore_axis_name="s")


def _user_gather(user_ids, uembT):
    info = plsc.get_sparse_core_info()
    nc, ns = info.num_cores, info.num_subcores
    bpw = B // (nc * ns)

    @functools.partial(
        pl.kernel,
        mesh=plsc.VectorSubcoreMesh(**_MESH),
        compiler_params=pltpu.CompilerParams(
            needs_layout_passes=False, disable_bounds_checks=True),
        out_type=jax.ShapeDtypeStruct((B * D,), jnp.float32),
        scratch_types=[
            pltpu.VMEM((bpw,), jnp.int32),
            *[pltpu.VMEM((D, 128), jnp.float32) for _ in range(L)],
            pltpu.VMEM((bpw * D,), jnp.float32),
            pltpu.SemaphoreType.DMA,
        ],
    )
    def ka(uids_hbm, uembT_hbm, out_hbm, uidx, *rest):
        ubuf = rest[:L]
        urows, usem = rest[L:]
        wid = lax.axis_index("s") * nc + lax.axis_index("c")
        base = wid * bpw

        pltpu.sync_copy(uids_hbm.at[pl.ds(base, bpw)], uidx)
        dlo = lax.iota(jnp.int32, L)
        dhi = dlo + L
        ngrp = bpw // L

        def wait_one(r):
            pltpu.make_async_copy(
                uembT_hbm.at[pl.ds(0, D), pl.ds(0, 128)], ubuf[r], usem).wait()

        def extract(uvec, g, r):
            ul = jnp.full((L,), uvec[r] & 127, jnp.int32)
            i = g * L + r
            urows[pl.ds(i * D, L)] = plsc.load_gather(ubuf[r], [dlo, ul])
            urows[pl.ds(i * D + L, L)] = plsc.load_gather(ubuf[r], [dhi, ul])

        def group(g, _):
            cur = uidx[pl.ds(g * L, L)]
            prv = uidx[pl.ds(jnp.maximum(g - 1, 0) * L, L)]
            for r in range(L):
                @pl.when(g > 0)
                def _():
                    wait_one(r)
                    extract(prv, g - 1, r)

                blk = pl.multiple_of((cur[r] >> 7) * 128, 128)
                pltpu.async_copy(
                    uembT_hbm.at[pl.ds(0, D), pl.ds(blk, 128)], ubuf[r], usem)
            return 0

        lax.fori_loop(0, ngrp, group, 0)
        lastv = uidx[pl.ds((ngrp - 1) * L, L)]
        for r in range(L):
            wait_one(r)
            extract(lastv, ngrp - 1, r)
        pltpu.sync_copy(urows, out_hbm.at[pl.ds(base * D, bpw * D)])

    return ka(user_ids, uembT)


def _dot_bias(user_rows_flat, user_ids, movie_ids, memb, ub1d, mb1d):
    info = plsc.get_sparse_core_info()
    nc, ns = info.num_cores, info.num_subcores
    bpw = B // (nc * ns)

    @functools.partial(
        pl.kernel,
        mesh=plsc.VectorSubcoreMesh(**_MESH),
        compiler_params=pltpu.CompilerParams(needs_layout_passes=False),
        out_type=jax.ShapeDtypeStruct((B,), jnp.float32),
        scratch_types=[
            pltpu.VMEM((bpw,), jnp.int32),
            pltpu.VMEM((bpw,), jnp.int32),
            *[pltpu.VMEM((8, D), jnp.float32) for _ in range(L)],
            pltpu.VMEM((bpw * D,), jnp.float32),
            pltpu.VMEM((bpw,), jnp.float32),
            pltpu.VMEM((bpw,), jnp.float32),
            pltpu.VMEM((bpw,), jnp.float32),
            pltpu.SemaphoreType.DMA,
            pltpu.SemaphoreType.DMA,
        ],
    )
    def kb(urows_hbm, uids_hbm, mids_hbm, memb_hbm, ub_hbm, mb_hbm, out_hbm,
           uidx, midx, *rest):
        mbuf = rest[:L]
        urows, ubias, mbias, outv, sem, msem = rest[L:]
        wid = lax.axis_index("s") * nc + lax.axis_index("c")
        base = wid * bpw

        pltpu.sync_copy(uids_hbm.at[pl.ds(base, bpw)], uidx)
        pltpu.sync_copy(mids_hbm.at[pl.ds(base, bpw)], midx)
        cu = pltpu.async_copy(
            urows_hbm.at[pl.ds(base * D, bpw * D)], urows, sem)
        cb1 = pltpu.async_copy(ub_hbm.at[uidx], ubias, sem)
        cb2 = pltpu.async_copy(mb_hbm.at[midx], mbias, sem)
        lane = lax.iota(jnp.int32, L)
        ngrp = bpw // L
        cu.wait()

        def wait_slab(r):
            pltpu.make_async_copy(
                memb_hbm.at[pl.ds(0, 8), pl.ds(0, D)], mbuf[r], msem).wait()

        def dot_one(mvec, g, r):
            i = g * L + r
            ms = mvec[r] & 7
            m0 = mbuf[r][ms, pl.ds(0, L)]
            m1 = mbuf[r][ms, pl.ds(L, L)]
            u0 = urows[pl.ds(i * D, L)]
            u1 = urows[pl.ds(i * D + L, L)]
            s = u0 * m0 + u1 * m1
            return jnp.sum(s)

        def group(g, _):
            gp = jnp.maximum(g - 1, 0)
            cur = midx[pl.ds(g * L, L)]
            prv = midx[pl.ds(gp * L, L)]
            acc = jnp.zeros((L,), jnp.float32)
            for r in range(L):
                @pl.when(g > 0)
                def _():
                    wait_slab(r)

                acc = acc + jnp.where(
                    (lane == r) & (g > 0), dot_one(prv, gp, r), 0.0)
                row8 = pl.multiple_of((cur[r] >> 3) * 8, 8)
                pltpu.async_copy(
                    memb_hbm.at[pl.ds(row8, 8), pl.ds(0, D)], mbuf[r], msem)

            @pl.when(g > 0)
            def _():
                outv[pl.ds(gp * L, L)] = acc

            return 0

        lax.fori_loop(0, ngrp, group, 0)
        lastv = midx[pl.ds((ngrp - 1) * L, L)]
        accl = jnp.zeros((L,), jnp.float32)
        for r in range(L):
            wait_slab(r)
            accl = accl + jnp.where(lane == r, dot_one(lastv, ngrp - 1, r), 0.0)
        outv[pl.ds((ngrp - 1) * L, L)] = accl

        cb1.wait()
        cb2.wait()

        def badd(j, _):
            sl = pl.ds(j * L, L)
            outv[sl] = outv[sl] + mbias[sl] + ubias[sl]
            return 0

        lax.fori_loop(0, bpw // L, badd, 0, unroll=4)

        pltpu.sync_copy(outv, out_hbm.at[pl.ds(base, bpw)])

    return kb(user_rows_flat, user_ids, movie_ids, memb, ub1d, mb1d)


def kernel(user_ids, movie_ids, user_emb, movie_emb, user_bias, movie_bias):
    n_movies, _ = movie_emb.shape
    uids = user_ids.astype(jnp.int32)
    mids = movie_ids.astype(jnp.int32)
    urows_flat = _user_gather(uids, user_emb.T)
    return _dot_bias(
        urows_flat,
        uids,
        mids,
        movie_emb,
        user_bias.reshape(-1),
        movie_bias.reshape(-1),
    )


# speedup vs baseline: 1.0336x; 1.0016x over previous
import functools

import jax
import jax.numpy as jnp
from jax import lax
from jax.experimental import pallas as pl
from jax.experimental.pallas import tpu as pltpu
from jax.experimental.pallas import tpu_sc as plsc

B = 16384
D = 32
L = 16

_MESH = dict(core_axis_name="c", subcore_axis_name="s")


def _user_gather(user_ids, uembT):
    info = plsc.get_sparse_core_info()
    nc, ns = info.num_cores, info.num_subcores
    bpw = B // (nc * ns)

    @functools.partial(
        pl.kernel,
        mesh=plsc.VectorSubcoreMesh(**_MESH),
        compiler_params=pltpu.CompilerParams(
            needs_layout_passes=False, disable_bounds_checks=True),
        out_type=jax.ShapeDtypeStruct((B * D,), jnp.float32),
        scratch_types=[
            pltpu.VMEM((bpw,), jnp.int32),
            *[pltpu.VMEM((D, 128), jnp.float32) for _ in range(L)],
            pltpu.VMEM((bpw * D,), jnp.float32),
            pltpu.SemaphoreType.DMA,
        ],
    )
    def ka(uids_hbm, uembT_hbm, out_hbm, uidx, *rest):
        ubuf = rest[:L]
        urows, usem = rest[L:]
        wid = lax.axis_index("s") * nc + lax.axis_index("c")
        base = wid * bpw

        pltpu.sync_copy(uids_hbm.at[pl.ds(base, bpw)], uidx)
        dlo = lax.iota(jnp.int32, L)
        dhi = dlo + L
        ngrp = bpw // L

        def wait_one(r):
            pltpu.make_async_copy(
                uembT_hbm.at[pl.ds(0, D), pl.ds(0, 128)], ubuf[r], usem).wait()

        def extract(uvec, g, r):
            ul = jnp.full((L,), uvec[r] & 127, jnp.int32)
            i = g * L + r
            urows[pl.ds(i * D, L)] = plsc.load_gather(ubuf[r], [dlo, ul])
            urows[pl.ds(i * D + L, L)] = plsc.load_gather(ubuf[r], [dhi, ul])

        def group(g, _):
            cur = uidx[pl.ds(g * L, L)]
            prv = uidx[pl.ds(jnp.maximum(g - 1, 0) * L, L)]
            for r in range(L):
                @pl.when(g > 0)
                def _():
                    wait_one(r)
                    extract(prv, g - 1, r)

                blk = pl.multiple_of((cur[r] >> 7) * 128, 128)
                pltpu.async_copy(
                    uembT_hbm.at[pl.ds(0, D), pl.ds(blk, 128)], ubuf[r], usem)
            return 0

        lax.fori_loop(0, ngrp, group, 0)
        lastv = uidx[pl.ds((ngrp - 1) * L, L)]
        for r in range(L):
            wait_one(r)
            extract(lastv, ngrp - 1, r)
        pltpu.sync_copy(urows, out_hbm.at[pl.ds(base * D, bpw * D)])

    return ka(user_ids, uembT)


def _dot_bias(user_rows_flat, user_ids, movie_ids, memb, ub1d, mb1d):
    info = plsc.get_sparse_core_info()
    nc, ns = info.num_cores, info.num_subcores
    bpw = B // (nc * ns)

    @functools.partial(
        pl.kernel,
        mesh=plsc.VectorSubcoreMesh(**_MESH),
        compiler_params=pltpu.CompilerParams(needs_layout_passes=False),
        out_type=jax.ShapeDtypeStruct((B,), jnp.float32),
        scratch_types=[
            pltpu.VMEM((bpw,), jnp.int32),
            pltpu.VMEM((bpw,), jnp.int32),
            *[pltpu.VMEM((8, D), jnp.float32) for _ in range(L)],
            pltpu.VMEM((bpw * D,), jnp.float32),
            pltpu.VMEM((bpw,), jnp.float32),
            pltpu.VMEM((bpw,), jnp.float32),
            pltpu.VMEM((bpw,), jnp.float32),
            pltpu.SemaphoreType.DMA,
            pltpu.SemaphoreType.DMA,
        ],
    )
    def kb(urows_hbm, uids_hbm, mids_hbm, memb_hbm, ub_hbm, mb_hbm, out_hbm,
           uidx, midx, *rest):
        mbuf = rest[:L]
        urows, ubias, mbias, outv, sem, msem = rest[L:]
        wid = lax.axis_index("s") * nc + lax.axis_index("c")
        base = wid * bpw

        pltpu.sync_copy(uids_hbm.at[pl.ds(base, bpw)], uidx)
        pltpu.sync_copy(mids_hbm.at[pl.ds(base, bpw)], midx)
        cu = pltpu.async_copy(
            urows_hbm.at[pl.ds(base * D, bpw * D)], urows, sem)
        cb1 = pltpu.async_copy(ub_hbm.at[uidx], ubias, sem)
        cb2 = pltpu.async_copy(mb_hbm.at[midx], mbias, sem)
        lane = lax.iota(jnp.int32, L)
        ngrp = bpw // L
        cu.wait()

        def wait_slab(r):
            pltpu.make_async_copy(
                memb_hbm.at[pl.ds(0, 8), pl.ds(0, D)], mbuf[r], msem).wait()

        def dot_one(mvec, g, r):
            i = g * L + r
            ms = mvec[r] & 7
            m0 = mbuf[r][ms, pl.ds(0, L)]
            m1 = mbuf[r][ms, pl.ds(L, L)]
            u0 = urows[pl.ds(i * D, L)]
            u1 = urows[pl.ds(i * D + L, L)]
            s = u0 * m0 + u1 * m1
            return jnp.sum(s)

        def group(g, _):
            gp = jnp.maximum(g - 1, 0)
            cur = midx[pl.ds(g * L, L)]
            prv = midx[pl.ds(gp * L, L)]
            acc = jnp.zeros((L,), jnp.float32)
            for r in range(L):
                @pl.when(g > 0)
                def _():
                    wait_slab(r)

                acc = acc + jnp.where(
                    (lane == r) & (g > 0), dot_one(prv, gp, r), 0.0)
                row8 = pl.multiple_of((cur[r] >> 3) * 8, 8)
                pltpu.async_copy(
                    memb_hbm.at[pl.ds(row8, 8), pl.ds(0, D)], mbuf[r], msem)

            @pl.when(g > 0)
            def _():
                outv[pl.ds(gp * L, L)] = acc

            return 0

        lax.fori_loop(0, ngrp, group, 0)
        lastv = midx[pl.ds((ngrp - 1) * L, L)]
        accl = jnp.zeros((L,), jnp.float32)
        for r in range(L):
            wait_slab(r)
            accl = accl + jnp.where(lane == r, dot_one(lastv, ngrp - 1, r), 0.0)
        outv[pl.ds((ngrp - 1) * L, L)] = accl

        cb1.wait()
        cb2.wait()

        def badd(j, _):
            sl = pl.ds(j * L, L)
            outv[sl] = outv[sl] + mbias[sl] + ubias[sl]
            return 0

        lax.fori_loop(0, bpw // L, badd, 0, unroll=4)

        pltpu.sync_copy(outv, out_hbm.at[pl.ds(base, bpw)])

    return kb(user_rows_flat, user_ids, movie_ids, memb, ub1d, mb1d)


def kernel(user_ids, movie_ids, user_emb, movie_emb, user_bias, movie_bias):
    uids = user_ids.astype(jnp.int32)
    mids = movie_ids.astype(jnp.int32)
    urows_flat = _user_gather(uids, user_emb.T)
    return _dot_bias(
        urows_flat,
        uids,
        mids,
        movie_emb,
        user_bias.reshape(-1),
        movie_bias.reshape(-1),
    )
